# baseline (speedup 1.0000x reference)
import functools

import numpy as np
import jax
import jax.numpy as jnp
from jax.experimental import pallas as pl
from jax.experimental.pallas import tpu as pltpu
from jax.experimental.shard_map import shard_map
from jax.sharding import Mesh, PartitionSpec as P

_LANES = 128
_S = 512
_W = 640


def _ceil_to(x, m):
    return -(-x // m) * m


def _sample_body(coarse_ref, comb_ref, seeds_ref, items_ref, prob_ref, *, k, r):
    seeds = seeds_ref[...]
    coarse = coarse_ref[...].reshape(1, 1, k)
    lt = (coarse < seeds[:, :, None]).astype(jnp.float32)
    b = jnp.minimum(jnp.sum(lt, axis=-1), float(k - 1))
    bi = b.astype(jnp.int32)
    iota = jax.lax.broadcasted_iota(jnp.int32, (r, _LANES, k), 2)
    onehot = (iota == bi[:, :, None]).astype(jnp.bfloat16)
    onehot2 = onehot.reshape(r * _LANES, k)

    def gat(j):
        g = jnp.dot(onehot2, comb_ref[:, j * _W:(j + 1) * _W],
                    preferred_element_type=jnp.float32)
        return g.reshape(r, _LANES, _W)

    t = (gat(0) + gat(1)) + gat(2)
    mask = (t < seeds[:, :, None]).astype(jnp.float32)
    val = gat(3)

    cnt = jnp.sum(mask, axis=-1) - 1.0
    psum = jnp.sum(mask * val, axis=-1)
    items_ref[...] = (b * _S + cnt).astype(jnp.int32)
    prob_ref[...] = psum


def _pos_body(comb_ref, items_ref, prob_ref, *, k, r):
    items = items_ref[...]
    b = jnp.minimum(items // _S, k - 1)
    local = items - b * _S
    iota = jax.lax.broadcasted_iota(jnp.int32, (r, _LANES, k), 2)
    onehot = (iota == b[:, :, None]).astype(jnp.bfloat16)
    onehot2 = onehot.reshape(r * _LANES, k)

    g = jnp.dot(onehot2, comb_ref[...], preferred_element_type=jnp.float32)
    val = g.reshape(r, _LANES, _W)
    ji = jax.lax.broadcasted_iota(jnp.int32, (1, 1, _W), 2)
    jcmp = jnp.where(ji == _S, -1, jnp.where(ji > _S, 1 << 30, ji))
    mask = (jcmp < local[:, :, None]).astype(jnp.float32)
    prob_ref[...] = jnp.sum(mask * val, axis=-1)


def _region(core, sentinel, padval, k):
    sent = jnp.full((k, 1), sentinel, jnp.float32)
    pad = jnp.full((k, _W - _S - 1), padval, jnp.float32)
    return jnp.concatenate([core, sent, pad], axis=1)


def _build_tables(table_pad, dlogp_pad):
    npad = table_pad.shape[-1]
    npb = _ceil_to(npad, _S)
    t = table_pad.reshape(-1)
    d = dlogp_pad.reshape(-1)
    if npb != npad:
        t = jnp.pad(t, (0, npb - npad), constant_values=2.0)
        d = jnp.pad(d, (0, npb - npad))
    k = npb // _S
    t2 = t.reshape(k, _S)
    d2 = d.reshape(k, _S)
    coarse = t2[:, -1].reshape(1, k)
    bsum = jnp.cumsum(jnp.sum(d2, axis=1))
    base = jnp.concatenate([jnp.zeros((1,), jnp.float32), bsum[:-1]])

    def trunc(x):
        u = jax.lax.bitcast_convert_type(x, jnp.int32)
        return jax.lax.bitcast_convert_type(u & jnp.int32(-65536), jnp.float32)

    def split3(x):
        h = trunc(x)
        r1 = x - h
        l1 = trunc(r1)
        l2 = r1 - l1
        return h, l1, l2

    def rnd(x):
        u = jax.lax.bitcast_convert_type(x, jnp.int32)
        return jax.lax.bitcast_convert_type(
            (u + jnp.int32(32768)) & jnp.int32(-65536), jnp.float32)

    th, tl1, tl2 = split3(t2)
    dh = rnd(d2)
    base_h = rnd(base)

    th_r = _region(th, -1.0, 2.0, k)
    tl1_r = _region(tl1, 0.0, 0.0, k)
    tl2_r = _region(tl2, 0.0, 0.0, k)
    vh_r = jnp.concatenate([dh, base_h[:, None],
                            jnp.zeros((k, _W - _S - 1), jnp.float32)], axis=1)
    comb = jnp.concatenate(
        [th_r, tl1_r, tl2_r, vh_r], axis=1).astype(jnp.bfloat16)
    comb_pos = vh_r.astype(jnp.bfloat16)
    return k, coarse, comb, comb_pos


def _tile_rows(flat, r):
    m = flat.shape[0]
    rows = max(1, _ceil_to(m, _LANES) // _LANES)
    rows_pad = _ceil_to(rows, r)
    total = rows_pad * _LANES
    if total != m:
        flat = jnp.pad(flat, (0, total - m))
    return flat.reshape(rows_pad, _LANES), rows_pad


def _device_mesh():
    devs = jax.devices()
    n = 2 if len(devs) >= 2 else 1
    return Mesh(np.array(devs[:n]), ("x",)), n


def kernel(table_pad, dlogp_pad, query, pos_items, seed_key):
    k, coarse, comb, comb_pos = _build_tables(table_pad, dlogp_pad)

    q_prefix = query.shape[:-1]
    num_queries = int(np.prod(q_prefix))
    num_neg = 32
    key = jax.random.wrap_key_data(seed_key)
    seeds = jax.random.uniform(key, (num_queries, num_neg), dtype=jnp.float32)

    mesh, ndev = _device_mesh()

    r = 32
    row_spec = pl.BlockSpec((r, _LANES), lambda i: (i, 0))
    m = num_queries * num_neg
    seeds2d, rows_pad = _tile_rows(seeds.reshape(-1), r * ndev)

    def sample_call(coarse_s, comb_s, seeds_s):
        rows = seeds_s.shape[0]
        return pl.pallas_call(
            functools.partial(_sample_body, k=k, r=r),
            out_shape=(jax.ShapeDtypeStruct((rows, _LANES), jnp.int32),
                       jax.ShapeDtypeStruct((rows, _LANES), jnp.float32)),
            grid=(rows // r,),
            in_specs=[pl.BlockSpec((1, k), lambda i: (0, 0)),
                      pl.BlockSpec((k, 4 * _W), lambda i: (0, 0)),
                      row_spec],
            out_specs=[row_spec, row_spec],
            compiler_params=pltpu.CompilerParams(
                dimension_semantics=("parallel",),
                vmem_limit_bytes=64 * 1024 * 1024),
        )(coarse_s, comb_s, seeds_s)

    if ndev > 1:
        sample_call = shard_map(
            sample_call, mesh=mesh,
            in_specs=(P(None, None), P(None, None), P("x", None)),
            out_specs=(P("x", None), P("x", None)), check_rep=False)
    items2d, prob2d = sample_call(coarse, comb, seeds2d)
    neg_items = items2d.reshape(-1)[:m].reshape(*q_prefix, num_neg)
    neg_prob = prob2d.reshape(-1)[:m].reshape(*q_prefix, num_neg)

    rp = 16
    prow_spec = pl.BlockSpec((rp, _LANES), lambda i: (i, 0))
    mp = int(np.prod(pos_items.shape))
    pos2d, prows_pad = _tile_rows(pos_items.reshape(-1).astype(jnp.int32),
                                  rp * ndev)

    def pos_call(comb_s, pos_s):
        rows = pos_s.shape[0]
        return pl.pallas_call(
            functools.partial(_pos_body, k=k, r=rp),
            out_shape=jax.ShapeDtypeStruct((rows, _LANES), jnp.float32),
            grid=(rows // rp,),
            in_specs=[pl.BlockSpec((k, _W), lambda i: (0, 0)), prow_spec],
            out_specs=prow_spec,
            compiler_params=pltpu.CompilerParams(
                dimension_semantics=("parallel",),
                vmem_limit_bytes=64 * 1024 * 1024),
        )(comb_s, pos_s)

    if ndev > 1:
        pos_call = shard_map(
            pos_call, mesh=mesh,
            in_specs=(P(None, None), P("x", None)),
            out_specs=P("x", None), check_rep=False)
    pprob2d = pos_call(comb_pos, pos2d)
    pos_prob = pprob2d.reshape(-1)[:mp].reshape(pos_items.shape)

    return pos_prob, neg_items, neg_prob


# speedup vs baseline: 1.0779x; 1.0779x over previous
import functools

import numpy as np
import jax
import jax.numpy as jnp
from jax.experimental import pallas as pl
from jax.experimental.pallas import tpu as pltpu
from jax.experimental.shard_map import shard_map
from jax.sharding import Mesh, PartitionSpec as P

_LANES = 128
_S = 512
_W = 640


def _ceil_to(x, m):
    return -(-x // m) * m


def _sample_body(coarse_ref, bs_ref, comb_ref, seeds_ref, items_ref, prob_ref,
                 *, k, r):
    seeds = seeds_ref[...]
    coarse = coarse_ref[...].reshape(1, 1, k)
    lt = (coarse < seeds[:, :, None]).astype(jnp.float32)
    b = jnp.minimum(jnp.sum(lt, axis=-1), float(k - 1))
    base = jnp.sum(lt * bs_ref[...].reshape(1, 1, k), axis=-1)
    bi = b.astype(jnp.int32)
    iota = jax.lax.broadcasted_iota(jnp.int32, (r, _LANES, k), 2)
    onehot = (iota == bi[:, :, None]).astype(jnp.bfloat16)
    onehot2 = onehot.reshape(r * _LANES, k)

    def gat(j):
        g = jnp.dot(onehot2, comb_ref[:, j * _S:(j + 1) * _S],
                    preferred_element_type=jnp.float32)
        return g.reshape(r, _LANES, _S)

    t = (gat(0) + gat(1)) + gat(2)
    mask = (t < seeds[:, :, None]).astype(jnp.float32)
    val = gat(3)

    cnt = jnp.sum(mask, axis=-1)
    psum = jnp.sum(mask * val, axis=-1) + base
    items_ref[...] = (b * _S + cnt).astype(jnp.int32)
    prob_ref[...] = psum


def _pos_body(bs_ref, comb_ref, items_ref, prob_ref, *, k, r):
    items = items_ref[...]
    b = jnp.minimum(items // _S, k - 1)
    local = items - b * _S
    iota = jax.lax.broadcasted_iota(jnp.int32, (r, _LANES, k), 2)
    ltp = (iota < b[:, :, None]).astype(jnp.float32)
    base = jnp.sum(ltp * bs_ref[...].reshape(1, 1, k), axis=-1)
    onehot = (iota == b[:, :, None]).astype(jnp.bfloat16)
    onehot2 = onehot.reshape(r * _LANES, k)

    g = jnp.dot(onehot2, comb_ref[...], preferred_element_type=jnp.float32)
    val = g.reshape(r, _LANES, _S)
    ji = jax.lax.broadcasted_iota(jnp.int32, (1, 1, _S), 2)
    mask = (ji < local[:, :, None]).astype(jnp.float32)
    prob_ref[...] = jnp.sum(mask * val, axis=-1) + base


def _region(core, sentinel, padval, k):
    sent = jnp.full((k, 1), sentinel, jnp.float32)
    pad = jnp.full((k, _W - _S - 1), padval, jnp.float32)
    return jnp.concatenate([core, sent, pad], axis=1)


def _build_tables(table_pad, dlogp_pad):
    npad = table_pad.shape[-1]
    npb = _ceil_to(npad, _S)
    t = table_pad.reshape(-1)
    d = dlogp_pad.reshape(-1)
    if npb != npad:
        t = jnp.pad(t, (0, npb - npad), constant_values=2.0)
        d = jnp.pad(d, (0, npb - npad))
    k = npb // _S
    t2 = t.reshape(k, _S)
    d2 = d.reshape(k, _S)
    coarse = t2[:, -1].reshape(1, k)
    bsum = jnp.cumsum(jnp.sum(d2, axis=1))
    base = jnp.concatenate([jnp.zeros((1,), jnp.float32), bsum[:-1]])

    def trunc(x):
        u = jax.lax.bitcast_convert_type(x, jnp.int32)
        return jax.lax.bitcast_convert_type(u & jnp.int32(-65536), jnp.float32)

    def split3(x):
        h = trunc(x)
        r1 = x - h
        l1 = trunc(r1)
        l2 = r1 - l1
        return h, l1, l2

    def rnd(x):
        u = jax.lax.bitcast_convert_type(x, jnp.int32)
        return jax.lax.bitcast_convert_type(
            (u + jnp.int32(32768)) & jnp.int32(-65536), jnp.float32)

    th, tl1, tl2 = split3(t2)
    dh = rnd(d2)

    bsums = jnp.sum(d2, axis=1).at[k - 1].set(0.0).reshape(1, k)

    comb = jnp.concatenate([th, tl1, tl2, dh], axis=1).astype(jnp.bfloat16)
    comb_pos = dh.astype(jnp.bfloat16)
    return k, coarse, bsums, comb, comb_pos


def _tile_rows(flat, r):
    m = flat.shape[0]
    rows = max(1, _ceil_to(m, _LANES) // _LANES)
    rows_pad = _ceil_to(rows, r)
    total = rows_pad * _LANES
    if total != m:
        flat = jnp.pad(flat, (0, total - m))
    return flat.reshape(rows_pad, _LANES), rows_pad


def _device_mesh():
    devs = jax.devices()
    n = 2 if len(devs) >= 2 else 1
    return Mesh(np.array(devs[:n]), ("x",)), n


def kernel(table_pad, dlogp_pad, query, pos_items, seed_key):
    k, coarse, bsums, comb, comb_pos = _build_tables(table_pad, dlogp_pad)

    q_prefix = query.shape[:-1]
    num_queries = int(np.prod(q_prefix))
    num_neg = 32
    key = jax.random.wrap_key_data(seed_key)
    seeds = jax.random.uniform(key, (num_queries, num_neg), dtype=jnp.float32)

    mesh, ndev = _device_mesh()

    r = 32
    row_spec = pl.BlockSpec((r, _LANES), lambda i: (i, 0))
    m = num_queries * num_neg
    seeds2d, rows_pad = _tile_rows(seeds.reshape(-1), r * ndev)

    def sample_call(coarse_s, bs_s, comb_s, seeds_s):
        rows = seeds_s.shape[0]
        return pl.pallas_call(
            functools.partial(_sample_body, k=k, r=r),
            out_shape=(jax.ShapeDtypeStruct((rows, _LANES), jnp.int32),
                       jax.ShapeDtypeStruct((rows, _LANES), jnp.float32)),
            grid=(rows // r,),
            in_specs=[pl.BlockSpec((1, k), lambda i: (0, 0)),
                      pl.BlockSpec((1, k), lambda i: (0, 0)),
                      pl.BlockSpec((k, 4 * _S), lambda i: (0, 0)),
                      row_spec],
            out_specs=[row_spec, row_spec],
            compiler_params=pltpu.CompilerParams(
                dimension_semantics=("parallel",),
                vmem_limit_bytes=64 * 1024 * 1024),
        )(coarse_s, bs_s, comb_s, seeds_s)

    if ndev > 1:
        sample_call = shard_map(
            sample_call, mesh=mesh,
            in_specs=(P(None, None), P(None, None), P(None, None),
                      P("x", None)),
            out_specs=(P("x", None), P("x", None)), check_rep=False)
    items2d, prob2d = sample_call(coarse, bsums, comb, seeds2d)
    neg_items = items2d.reshape(-1)[:m].reshape(*q_prefix, num_neg)
    neg_prob = prob2d.reshape(-1)[:m].reshape(*q_prefix, num_neg)

    rp = 16
    prow_spec = pl.BlockSpec((rp, _LANES), lambda i: (i, 0))
    mp = int(np.prod(pos_items.shape))
    pos2d, prows_pad = _tile_rows(pos_items.reshape(-1).astype(jnp.int32),
                                  rp * ndev)

    def pos_call(bs_s, comb_s, pos_s):
        rows = pos_s.shape[0]
        return pl.pallas_call(
            functools.partial(_pos_body, k=k, r=rp),
            out_shape=jax.ShapeDtypeStruct((rows, _LANES), jnp.float32),
            grid=(rows // rp,),
            in_specs=[pl.BlockSpec((1, k), lambda i: (0, 0)),
                      pl.BlockSpec((k, _S), lambda i: (0, 0)), prow_spec],
            out_specs=prow_spec,
            compiler_params=pltpu.CompilerParams(
                dimension_semantics=("parallel",),
                vmem_limit_bytes=64 * 1024 * 1024),
        )(bs_s, comb_s, pos_s)

    if ndev > 1:
        pos_call = shard_map(
            pos_call, mesh=mesh,
            in_specs=(P(None, None), P(None, None), P("x", None)),
            out_specs=P("x", None), check_rep=False)
    pprob2d = pos_call(bsums, comb_pos, pos2d)
    pos_prob = pprob2d.reshape(-1)[:mp].reshape(pos_items.shape)

    return pos_prob, neg_items, neg_prob


# speedup vs baseline: 1.1656x; 1.0814x over previous
import functools

import numpy as np
import jax
import jax.numpy as jnp
from jax.experimental import pallas as pl
from jax.experimental.pallas import tpu as pltpu
from jax.experimental.shard_map import shard_map
from jax.sharding import Mesh, PartitionSpec as P

_LANES = 128
_S = 512
_W = 640


def _ceil_to(x, m):
    return -(-x // m) * m


def _sample_half(coarse_ref, bs_ref, comb_ref, seeds, *, k, r):
    coarse = coarse_ref[...].reshape(1, 1, k)
    lt = (coarse < seeds[:, :, None]).astype(jnp.float32)
    b = jnp.minimum(jnp.sum(lt, axis=-1), float(k - 1))
    base = jnp.sum(lt * bs_ref[...].reshape(1, 1, k), axis=-1)
    bi = b.astype(jnp.int32)
    iota = jax.lax.broadcasted_iota(jnp.int32, (r, _LANES, k), 2)
    onehot = (iota == bi[:, :, None]).astype(jnp.bfloat16)
    onehot2 = onehot.reshape(r * _LANES, k)

    def gat(j):
        g = jnp.dot(onehot2, comb_ref[:, j * _S:(j + 1) * _S],
                    preferred_element_type=jnp.float32)
        return g.reshape(r, _LANES, _S)

    t = (gat(0) + gat(1)) + gat(2)
    mask = (t < seeds[:, :, None]).astype(jnp.float32)
    val = gat(3)

    cnt = jnp.sum(mask, axis=-1)
    psum = jnp.sum(mask * val, axis=-1) + base
    return (b * _S + cnt).astype(jnp.int32), psum


def _sample_body(coarse_ref, bs_ref, comb_ref, seeds_ref, items_ref, prob_ref,
                 *, k, r, halves):
    h = r // halves
    for i in range(halves):
        sl = slice(i * h, (i + 1) * h)
        items, prob = _sample_half(coarse_ref, bs_ref, comb_ref,
                                   seeds_ref[sl, :], k=k, r=h)
        items_ref[sl, :] = items
        prob_ref[sl, :] = prob


def _pos_body(bs_ref, comb_ref, items_ref, prob_ref, *, k, r):
    items = items_ref[...]
    b = jnp.minimum(items // _S, k - 1)
    local = items - b * _S
    iota = jax.lax.broadcasted_iota(jnp.int32, (r, _LANES, k), 2)
    ltp = (iota < b[:, :, None]).astype(jnp.float32)
    base = jnp.sum(ltp * bs_ref[...].reshape(1, 1, k), axis=-1)
    onehot = (iota == b[:, :, None]).astype(jnp.bfloat16)
    onehot2 = onehot.reshape(r * _LANES, k)

    g = jnp.dot(onehot2, comb_ref[...], preferred_element_type=jnp.float32)
    val = g.reshape(r, _LANES, _S)
    ji = jax.lax.broadcasted_iota(jnp.int32, (1, 1, _S), 2)
    mask = (ji < local[:, :, None]).astype(jnp.float32)
    prob_ref[...] = jnp.sum(mask * val, axis=-1) + base


def _region(core, sentinel, padval, k):
    sent = jnp.full((k, 1), sentinel, jnp.float32)
    pad = jnp.full((k, _W - _S - 1), padval, jnp.float32)
    return jnp.concatenate([core, sent, pad], axis=1)


def _build_tables(table_pad, dlogp_pad):
    npad = table_pad.shape[-1]
    npb = _ceil_to(npad, _S)
    t = table_pad.reshape(-1)
    d = dlogp_pad.reshape(-1)
    if npb != npad:
        t = jnp.pad(t, (0, npb - npad), constant_values=2.0)
        d = jnp.pad(d, (0, npb - npad))
    k = npb // _S
    t2 = t.reshape(k, _S)
    d2 = d.reshape(k, _S)
    coarse = t2[:, -1].reshape(1, k)
    bsum = jnp.cumsum(jnp.sum(d2, axis=1))
    base = jnp.concatenate([jnp.zeros((1,), jnp.float32), bsum[:-1]])

    def trunc(x):
        u = jax.lax.bitcast_convert_type(x, jnp.int32)
        return jax.lax.bitcast_convert_type(u & jnp.int32(-65536), jnp.float32)

    def split3(x):
        h = trunc(x)
        r1 = x - h
        l1 = trunc(r1)
        l2 = r1 - l1
        return h, l1, l2

    def rnd(x):
        u = jax.lax.bitcast_convert_type(x, jnp.int32)
        return jax.lax.bitcast_convert_type(
            (u + jnp.int32(32768)) & jnp.int32(-65536), jnp.float32)

    th, tl1, tl2 = split3(t2)
    dh = rnd(d2)

    bsums = jnp.sum(d2, axis=1).at[k - 1].set(0.0).reshape(1, k)

    comb = jnp.concatenate([th, tl1, tl2, dh], axis=1).astype(jnp.bfloat16)
    comb_pos = dh.astype(jnp.bfloat16)
    return k, coarse, bsums, comb, comb_pos


def _tile_rows(flat, r):
    m = flat.shape[0]
    rows = max(1, _ceil_to(m, _LANES) // _LANES)
    rows_pad = _ceil_to(rows, r)
    total = rows_pad * _LANES
    if total != m:
        flat = jnp.pad(flat, (0, total - m))
    return flat.reshape(rows_pad, _LANES), rows_pad


def _device_mesh():
    devs = jax.devices()
    n = 2 if len(devs) >= 2 else 1
    return Mesh(np.array(devs[:n]), ("x",)), n


def kernel(table_pad, dlogp_pad, query, pos_items, seed_key):
    k, coarse, bsums, comb, comb_pos = _build_tables(table_pad, dlogp_pad)

    q_prefix = query.shape[:-1]
    num_queries = int(np.prod(q_prefix))
    num_neg = 32
    key = jax.random.wrap_key_data(seed_key)
    seeds = jax.random.uniform(key, (num_queries, num_neg), dtype=jnp.float32)

    mesh, ndev = _device_mesh()

    r = 32
    row_spec = pl.BlockSpec((r, _LANES), lambda i: (i, 0))
    m = num_queries * num_neg
    seeds2d, rows_pad = _tile_rows(seeds.reshape(-1), r * ndev)

    def sample_call(coarse_s, bs_s, comb_s, seeds_s):
        rows = seeds_s.shape[0]
        return pl.pallas_call(
            functools.partial(_sample_body, k=k, r=r, halves=2),
            out_shape=(jax.ShapeDtypeStruct((rows, _LANES), jnp.int32),
                       jax.ShapeDtypeStruct((rows, _LANES), jnp.float32)),
            grid=(rows // r,),
            in_specs=[pl.BlockSpec((1, k), lambda i: (0, 0)),
                      pl.BlockSpec((1, k), lambda i: (0, 0)),
                      pl.BlockSpec((k, 4 * _S), lambda i: (0, 0)),
                      row_spec],
            out_specs=[row_spec, row_spec],
            compiler_params=pltpu.CompilerParams(
                dimension_semantics=("parallel",),
                vmem_limit_bytes=64 * 1024 * 1024),
        )(coarse_s, bs_s, comb_s, seeds_s)

    if ndev > 1:
        sample_call = shard_map(
            sample_call, mesh=mesh,
            in_specs=(P(None, None), P(None, None), P(None, None),
                      P("x", None)),
            out_specs=(P("x", None), P("x", None)), check_rep=False)
    items2d, prob2d = sample_call(coarse, bsums, comb, seeds2d)
    neg_items = items2d.reshape(-1)[:m].reshape(*q_prefix, num_neg)
    neg_prob = prob2d.reshape(-1)[:m].reshape(*q_prefix, num_neg)

    rp = 16
    prow_spec = pl.BlockSpec((rp, _LANES), lambda i: (i, 0))
    mp = int(np.prod(pos_items.shape))
    pos2d, prows_pad = _tile_rows(pos_items.reshape(-1).astype(jnp.int32),
                                  rp * ndev)

    def pos_call(bs_s, comb_s, pos_s):
        rows = pos_s.shape[0]
        return pl.pallas_call(
            functools.partial(_pos_body, k=k, r=rp),
            out_shape=jax.ShapeDtypeStruct((rows, _LANES), jnp.float32),
            grid=(rows // rp,),
            in_specs=[pl.BlockSpec((1, k), lambda i: (0, 0)),
                      pl.BlockSpec((k, _S), lambda i: (0, 0)), prow_spec],
            out_specs=prow_spec,
            compiler_params=pltpu.CompilerParams(
                dimension_semantics=("parallel",),
                vmem_limit_bytes=64 * 1024 * 1024),
        )(bs_s, comb_s, pos_s)

    if ndev > 1:
        pos_call = shard_map(
            pos_call, mesh=mesh,
            in_specs=(P(None, None), P(None, None), P("x", None)),
            out_specs=P("x", None), check_rep=False)
    pprob2d = pos_call(bsums, comb_pos, pos2d)
    pos_prob = pprob2d.reshape(-1)[:mp].reshape(pos_items.shape)

    return pos_prob, neg_items, neg_prob


# speedup vs baseline: 1.2519x; 1.0740x over previous
import functools

import numpy as np
import jax
import jax.numpy as jnp
from jax.experimental import pallas as pl
from jax.experimental.pallas import tpu as pltpu
from jax.experimental.shard_map import shard_map
from jax.sharding import Mesh, PartitionSpec as P

_LANES = 128
_S = 512
_W = 640


def _ceil_to(x, m):
    return -(-x // m) * m


def _sample_half(coarse_ref, bs_ref, comb_ref, seeds, *, k, r):
    coarse = coarse_ref[...].reshape(1, 1, k)
    lt = (coarse < seeds[:, :, None]).astype(jnp.float32)
    b = jnp.minimum(jnp.sum(lt, axis=-1), float(k - 1))
    base = jnp.sum(lt * bs_ref[...].reshape(1, 1, k), axis=-1)
    bi = b.astype(jnp.int32)
    iota = jax.lax.broadcasted_iota(jnp.int32, (r, _LANES, k), 2)
    onehot = (iota == bi[:, :, None]).astype(jnp.bfloat16)
    onehot2 = onehot.reshape(r * _LANES, k)

    def gat(j):
        g = jnp.dot(onehot2, comb_ref[:, j * _S:(j + 1) * _S],
                    preferred_element_type=jnp.float32)
        return g.reshape(r, _LANES, _S)

    t = (gat(0) + gat(1)) + gat(2)
    mask = (t < seeds[:, :, None]).astype(jnp.float32)
    val = gat(3)

    cnt = jnp.sum(mask, axis=-1)
    psum = jnp.sum(mask * val, axis=-1) + base
    return (b * _S + cnt).astype(jnp.int32), psum


def _sample_body(coarse_ref, bs_ref, comb_ref, seeds_ref, items_ref, prob_ref,
                 *, k, r, halves):
    h = r // halves
    for i in range(halves):
        sl = slice(i * h, (i + 1) * h)
        items, prob = _sample_half(coarse_ref, bs_ref, comb_ref,
                                   seeds_ref[sl, :], k=k, r=h)
        items_ref[sl, :] = items
        prob_ref[sl, :] = prob


def _pos_body(bs_ref, comb_ref, items_ref, prob_ref, *, k, r):
    items = items_ref[...]
    b = jnp.minimum(items // _S, k - 1)
    local = items - b * _S
    iota = jax.lax.broadcasted_iota(jnp.int32, (r, _LANES, k), 2)
    ltp = (iota < b[:, :, None]).astype(jnp.float32)
    base = jnp.sum(ltp * bs_ref[...].reshape(1, 1, k), axis=-1)
    onehot = (iota == b[:, :, None]).astype(jnp.bfloat16)
    onehot2 = onehot.reshape(r * _LANES, k)

    g = jnp.dot(onehot2, comb_ref[...], preferred_element_type=jnp.float32)
    val = g.reshape(r, _LANES, _S)
    ji = jax.lax.broadcasted_iota(jnp.int32, (1, 1, _S), 2)
    mask = (ji < local[:, :, None]).astype(jnp.float32)
    prob_ref[...] = jnp.sum(mask * val, axis=-1) + base


def _region(core, sentinel, padval, k):
    sent = jnp.full((k, 1), sentinel, jnp.float32)
    pad = jnp.full((k, _W - _S - 1), padval, jnp.float32)
    return jnp.concatenate([core, sent, pad], axis=1)


def _build_tables(table_pad, dlogp_pad):
    npad = table_pad.shape[-1]
    npb = _ceil_to(npad, _S)
    t = table_pad.reshape(-1)
    d = dlogp_pad.reshape(-1)
    if npb != npad:
        t = jnp.pad(t, (0, npb - npad), constant_values=2.0)
        d = jnp.pad(d, (0, npb - npad))
    k = npb // _S
    t2 = t.reshape(k, _S)
    d2 = d.reshape(k, _S)
    coarse = t2[:, -1].reshape(1, k)
    bsum = jnp.cumsum(jnp.sum(d2, axis=1))
    base = jnp.concatenate([jnp.zeros((1,), jnp.float32), bsum[:-1]])

    def trunc(x):
        u = jax.lax.bitcast_convert_type(x, jnp.int32)
        return jax.lax.bitcast_convert_type(u & jnp.int32(-65536), jnp.float32)

    def split3(x):
        h = trunc(x)
        r1 = x - h
        l1 = trunc(r1)
        l2 = r1 - l1
        return h, l1, l2

    def rnd(x):
        u = jax.lax.bitcast_convert_type(x, jnp.int32)
        return jax.lax.bitcast_convert_type(
            (u + jnp.int32(32768)) & jnp.int32(-65536), jnp.float32)

    th, tl1, tl2 = split3(t2)
    dh = rnd(d2)

    bsums = jnp.sum(d2, axis=1).at[k - 1].set(0.0).reshape(1, k)

    comb = jnp.concatenate([th, tl1, tl2, dh], axis=1).astype(jnp.bfloat16)
    comb_pos = dh.astype(jnp.bfloat16)
    return k, coarse, bsums, comb, comb_pos


def _tile_rows(flat, r):
    m = flat.shape[0]
    rows = max(1, _ceil_to(m, _LANES) // _LANES)
    rows_pad = _ceil_to(rows, r)
    total = rows_pad * _LANES
    if total != m:
        flat = jnp.pad(flat, (0, total - m))
    return flat.reshape(rows_pad, _LANES), rows_pad


def _device_mesh():
    devs = jax.devices()
    n = 2 if len(devs) >= 2 else 1
    return Mesh(np.array(devs[:n]), ("x",)), n


def kernel(table_pad, dlogp_pad, query, pos_items, seed_key):
    k, coarse, bsums, comb, comb_pos = _build_tables(table_pad, dlogp_pad)

    q_prefix = query.shape[:-1]
    num_queries = int(np.prod(q_prefix))
    num_neg = 32
    key = jax.random.wrap_key_data(seed_key)
    seeds = jax.random.uniform(key, (num_queries, num_neg), dtype=jnp.float32)

    mesh, ndev = _device_mesh()

    r = 32
    row_spec = pl.BlockSpec((r, _LANES), lambda i: (i, 0))
    m = num_queries * num_neg
    seeds2d, rows_pad = _tile_rows(seeds.reshape(-1), r * ndev)

    def sample_call(coarse_s, bs_s, comb_s, seeds_s):
        rows = seeds_s.shape[0]
        return pl.pallas_call(
            functools.partial(_sample_body, k=k, r=r, halves=4),
            out_shape=(jax.ShapeDtypeStruct((rows, _LANES), jnp.int32),
                       jax.ShapeDtypeStruct((rows, _LANES), jnp.float32)),
            grid=(rows // r,),
            in_specs=[pl.BlockSpec((1, k), lambda i: (0, 0)),
                      pl.BlockSpec((1, k), lambda i: (0, 0)),
                      pl.BlockSpec((k, 4 * _S), lambda i: (0, 0)),
                      row_spec],
            out_specs=[row_spec, row_spec],
            compiler_params=pltpu.CompilerParams(
                dimension_semantics=("parallel",),
                vmem_limit_bytes=64 * 1024 * 1024),
        )(coarse_s, bs_s, comb_s, seeds_s)

    if ndev > 1:
        sample_call = shard_map(
            sample_call, mesh=mesh,
            in_specs=(P(None, None), P(None, None), P(None, None),
                      P("x", None)),
            out_specs=(P("x", None), P("x", None)), check_rep=False)
    items2d, prob2d = sample_call(coarse, bsums, comb, seeds2d)
    neg_items = items2d.reshape(-1)[:m].reshape(*q_prefix, num_neg)
    neg_prob = prob2d.reshape(-1)[:m].reshape(*q_prefix, num_neg)

    rp = 16
    prow_spec = pl.BlockSpec((rp, _LANES), lambda i: (i, 0))
    mp = int(np.prod(pos_items.shape))
    pos2d, prows_pad = _tile_rows(pos_items.reshape(-1).astype(jnp.int32),
                                  rp * ndev)

    def pos_call(bs_s, comb_s, pos_s):
        rows = pos_s.shape[0]
        return pl.pallas_call(
            functools.partial(_pos_body, k=k, r=rp),
            out_shape=jax.ShapeDtypeStruct((rows, _LANES), jnp.float32),
            grid=(rows // rp,),
            in_specs=[pl.BlockSpec((1, k), lambda i: (0, 0)),
                      pl.BlockSpec((k, _S), lambda i: (0, 0)), prow_spec],
            out_specs=prow_spec,
            compiler_params=pltpu.CompilerParams(
                dimension_semantics=("parallel",),
                vmem_limit_bytes=64 * 1024 * 1024),
        )(bs_s, comb_s, pos_s)

    if ndev > 1:
        pos_call = shard_map(
            pos_call, mesh=mesh,
            in_specs=(P(None, None), P(None, None), P("x", None)),
            out_specs=P("x", None), check_rep=False)
    pprob2d = pos_call(bsums, comb_pos, pos2d)
    pos_prob = pprob2d.reshape(-1)[:mp].reshape(pos_items.shape)

    return pos_prob, neg_items, neg_prob


# speedup vs baseline: 1.2915x; 1.0317x over previous
import functools

import numpy as np
import jax
import jax.numpy as jnp
from jax.experimental import pallas as pl
from jax.experimental.pallas import tpu as pltpu
from jax.experimental.shard_map import shard_map
from jax.sharding import Mesh, PartitionSpec as P

_LANES = 128
_S = 512
_W = 640


def _ceil_to(x, m):
    return -(-x // m) * m


def _sample_half(coarse_ref, bs_ref, comb_ref, seeds, *, k, r):
    coarse = coarse_ref[...].reshape(1, 1, k)
    lt = (coarse < seeds[:, :, None]).astype(jnp.float32)
    b = jnp.minimum(jnp.sum(lt, axis=-1), float(k - 1))
    base = jnp.sum(lt * bs_ref[...].reshape(1, 1, k), axis=-1)
    bi = b.astype(jnp.int32)
    iota = jax.lax.broadcasted_iota(jnp.int32, (r, _LANES, k), 2)
    onehot = (iota == bi[:, :, None]).astype(jnp.bfloat16)
    onehot2 = onehot.reshape(r * _LANES, k)

    def gat(j):
        g = jnp.dot(onehot2, comb_ref[:, j * _S:(j + 1) * _S],
                    preferred_element_type=jnp.float32)
        return g.reshape(r, _LANES, _S)

    t = (gat(0) + gat(1)) + gat(2)
    mask = (t < seeds[:, :, None]).astype(jnp.float32)
    val = gat(3)

    cnt = jnp.sum(mask, axis=-1)
    psum = jnp.sum(mask * val, axis=-1) + base
    return (b * _S + cnt).astype(jnp.int32), psum


def _sample_body(coarse_ref, bs_ref, comb_ref, seeds_ref, items_ref, prob_ref,
                 *, k, r, halves):
    h = r // halves
    for i in range(halves):
        sl = slice(i * h, (i + 1) * h)
        items, prob = _sample_half(coarse_ref, bs_ref, comb_ref,
                                   seeds_ref[sl, :], k=k, r=h)
        items_ref[sl, :] = items
        prob_ref[sl, :] = prob


def _pos_body(bs_ref, comb_ref, items_ref, prob_ref, *, k, r):
    items = items_ref[...]
    b = jnp.minimum(items // _S, k - 1)
    local = items - b * _S
    iota = jax.lax.broadcasted_iota(jnp.int32, (r, _LANES, k), 2)
    ltp = (iota < b[:, :, None]).astype(jnp.float32)
    base = jnp.sum(ltp * bs_ref[...].reshape(1, 1, k), axis=-1)
    onehot = (iota == b[:, :, None]).astype(jnp.bfloat16)
    onehot2 = onehot.reshape(r * _LANES, k)

    g = jnp.dot(onehot2, comb_ref[...], preferred_element_type=jnp.float32)
    val = g.reshape(r, _LANES, _S)
    ji = jax.lax.broadcasted_iota(jnp.int32, (1, 1, _S), 2)
    mask = (ji < local[:, :, None]).astype(jnp.float32)
    prob_ref[...] = jnp.sum(mask * val, axis=-1) + base


def _region(core, sentinel, padval, k):
    sent = jnp.full((k, 1), sentinel, jnp.float32)
    pad = jnp.full((k, _W - _S - 1), padval, jnp.float32)
    return jnp.concatenate([core, sent, pad], axis=1)


def _build_tables(table_pad, dlogp_pad):
    npad = table_pad.shape[-1]
    npb = _ceil_to(npad, _S)
    t = table_pad.reshape(-1)
    d = dlogp_pad.reshape(-1)
    if npb != npad:
        t = jnp.pad(t, (0, npb - npad), constant_values=2.0)
        d = jnp.pad(d, (0, npb - npad))
    k = npb // _S
    t2 = t.reshape(k, _S)
    d2 = d.reshape(k, _S)
    coarse = t2[:, -1].reshape(1, k)
    bsum = jnp.cumsum(jnp.sum(d2, axis=1))
    base = jnp.concatenate([jnp.zeros((1,), jnp.float32), bsum[:-1]])

    def trunc(x):
        u = jax.lax.bitcast_convert_type(x, jnp.int32)
        return jax.lax.bitcast_convert_type(u & jnp.int32(-65536), jnp.float32)

    def split3(x):
        h = trunc(x)
        r1 = x - h
        l1 = trunc(r1)
        l2 = r1 - l1
        return h, l1, l2

    def rnd(x):
        u = jax.lax.bitcast_convert_type(x, jnp.int32)
        return jax.lax.bitcast_convert_type(
            (u + jnp.int32(32768)) & jnp.int32(-65536), jnp.float32)

    th, tl1, tl2 = split3(t2)
    dh = rnd(d2)

    bsums = jnp.sum(d2, axis=1).at[k - 1].set(0.0).reshape(1, k)

    comb = jnp.concatenate([th, tl1, tl2, dh], axis=1).astype(jnp.bfloat16)
    comb_pos = dh.astype(jnp.bfloat16)
    return k, coarse, bsums, comb, comb_pos


def _tile_rows(flat, r):
    m = flat.shape[0]
    rows = max(1, _ceil_to(m, _LANES) // _LANES)
    rows_pad = _ceil_to(rows, r)
    total = rows_pad * _LANES
    if total != m:
        flat = jnp.pad(flat, (0, total - m))
    return flat.reshape(rows_pad, _LANES), rows_pad


def _device_mesh():
    devs = jax.devices()
    n = 2 if len(devs) >= 2 else 1
    return Mesh(np.array(devs[:n]), ("x",)), n


def kernel(table_pad, dlogp_pad, query, pos_items, seed_key):
    k, coarse, bsums, comb, comb_pos = _build_tables(table_pad, dlogp_pad)

    q_prefix = query.shape[:-1]
    num_queries = int(np.prod(q_prefix))
    num_neg = 32
    key = jax.random.wrap_key_data(seed_key)
    seeds = jax.random.uniform(key, (num_queries, num_neg), dtype=jnp.float32)

    mesh, ndev = _device_mesh()

    r = 32
    row_spec = pl.BlockSpec((r, _LANES), lambda i: (i, 0))
    m = num_queries * num_neg
    seeds2d, rows_pad = _tile_rows(seeds.reshape(-1), r * ndev)

    def sample_call(coarse_s, bs_s, comb_s, seeds_s):
        rows = seeds_s.shape[0]
        return pl.pallas_call(
            functools.partial(_sample_body, k=k, r=r, halves=8),
            out_shape=(jax.ShapeDtypeStruct((rows, _LANES), jnp.int32),
                       jax.ShapeDtypeStruct((rows, _LANES), jnp.float32)),
            grid=(rows // r,),
            in_specs=[pl.BlockSpec((1, k), lambda i: (0, 0)),
                      pl.BlockSpec((1, k), lambda i: (0, 0)),
                      pl.BlockSpec((k, 4 * _S), lambda i: (0, 0)),
                      row_spec],
            out_specs=[row_spec, row_spec],
            compiler_params=pltpu.CompilerParams(
                dimension_semantics=("parallel",),
                vmem_limit_bytes=64 * 1024 * 1024),
        )(coarse_s, bs_s, comb_s, seeds_s)

    if ndev > 1:
        sample_call = shard_map(
            sample_call, mesh=mesh,
            in_specs=(P(None, None), P(None, None), P(None, None),
                      P("x", None)),
            out_specs=(P("x", None), P("x", None)), check_rep=False)
    items2d, prob2d = sample_call(coarse, bsums, comb, seeds2d)
    neg_items = items2d.reshape(-1)[:m].reshape(*q_prefix, num_neg)
    neg_prob = prob2d.reshape(-1)[:m].reshape(*q_prefix, num_neg)

    rp = 16
    prow_spec = pl.BlockSpec((rp, _LANES), lambda i: (i, 0))
    mp = int(np.prod(pos_items.shape))
    pos2d, prows_pad = _tile_rows(pos_items.reshape(-1).astype(jnp.int32),
                                  rp * ndev)

    def pos_call(bs_s, comb_s, pos_s):
        rows = pos_s.shape[0]
        return pl.pallas_call(
            functools.partial(_pos_body, k=k, r=rp),
            out_shape=jax.ShapeDtypeStruct((rows, _LANES), jnp.float32),
            grid=(rows // rp,),
            in_specs=[pl.BlockSpec((1, k), lambda i: (0, 0)),
                      pl.BlockSpec((k, _S), lambda i: (0, 0)), prow_spec],
            out_specs=prow_spec,
            compiler_params=pltpu.CompilerParams(
                dimension_semantics=("parallel",),
                vmem_limit_bytes=64 * 1024 * 1024),
        )(bs_s, comb_s, pos_s)

    if ndev > 1:
        pos_call = shard_map(
            pos_call, mesh=mesh,
            in_specs=(P(None, None), P(None, None), P("x", None)),
            out_specs=P("x", None), check_rep=False)
    pprob2d = pos_call(bsums, comb_pos, pos2d)
    pos_prob = pprob2d.reshape(-1)[:mp].reshape(pos_items.shape)

    return pos_prob, neg_items, neg_prob


# speedup vs baseline: 1.3289x; 1.0290x over previous
import functools

import numpy as np
import jax
import jax.numpy as jnp
from jax.experimental import pallas as pl
from jax.experimental.pallas import tpu as pltpu
from jax.experimental.shard_map import shard_map
from jax.sharding import Mesh, PartitionSpec as P

_LANES = 128
_S = 512
_W = 640


def _ceil_to(x, m):
    return -(-x // m) * m


def _sample_half(coarse_ref, bs_ref, comb_ref, seeds, *, k, r):
    coarse = coarse_ref[...].reshape(1, 1, k)
    lt = (coarse < seeds[:, :, None]).astype(jnp.float32)
    b = jnp.minimum(jnp.sum(lt, axis=-1), float(k - 1))
    base = jnp.sum(lt * bs_ref[...].reshape(1, 1, k), axis=-1)
    bi = b.astype(jnp.int32)
    iota = jax.lax.broadcasted_iota(jnp.int32, (r, _LANES, k), 2)
    onehot = (iota == bi[:, :, None]).astype(jnp.bfloat16)
    onehot2 = onehot.reshape(r * _LANES, k)

    def gat(j):
        g = jnp.dot(onehot2, comb_ref[:, j * _S:(j + 1) * _S],
                    preferred_element_type=jnp.float32)
        return g.reshape(r, _LANES, _S)

    t = (gat(0) + gat(1)) + gat(2)
    mask = (t < seeds[:, :, None]).astype(jnp.float32)
    val = gat(3)

    cnt = jnp.sum(mask, axis=-1)
    psum = jnp.sum(mask * val, axis=-1) + base
    return (b * _S + cnt).astype(jnp.int32), psum


def _sample_body(coarse_ref, bs_ref, comb_ref, seeds_ref, items_ref, prob_ref,
                 *, k, r, halves):
    h = r // halves
    for i in range(halves):
        sl = slice(i * h, (i + 1) * h)
        items, prob = _sample_half(coarse_ref, bs_ref, comb_ref,
                                   seeds_ref[sl, :], k=k, r=h)
        items_ref[sl, :] = items
        prob_ref[sl, :] = prob


def _pos_body(bs_ref, comb_ref, items_ref, prob_ref, *, k, r):
    items = items_ref[...]
    b = jnp.minimum(items // _S, k - 1)
    local = items - b * _S
    iota = jax.lax.broadcasted_iota(jnp.int32, (r, _LANES, k), 2)
    ltp = (iota < b[:, :, None]).astype(jnp.float32)
    base = jnp.sum(ltp * bs_ref[...].reshape(1, 1, k), axis=-1)
    onehot = (iota == b[:, :, None]).astype(jnp.bfloat16)
    onehot2 = onehot.reshape(r * _LANES, k)

    g = jnp.dot(onehot2, comb_ref[...], preferred_element_type=jnp.float32)
    val = g.reshape(r, _LANES, _S)
    ji = jax.lax.broadcasted_iota(jnp.int32, (1, 1, _S), 2)
    mask = (ji < local[:, :, None]).astype(jnp.float32)
    prob_ref[...] = jnp.sum(mask * val, axis=-1) + base


def _region(core, sentinel, padval, k):
    sent = jnp.full((k, 1), sentinel, jnp.float32)
    pad = jnp.full((k, _W - _S - 1), padval, jnp.float32)
    return jnp.concatenate([core, sent, pad], axis=1)


def _build_tables(table_pad, dlogp_pad):
    npad = table_pad.shape[-1]
    npb = _ceil_to(npad, _S)
    t = table_pad.reshape(-1)
    d = dlogp_pad.reshape(-1)
    if npb != npad:
        t = jnp.pad(t, (0, npb - npad), constant_values=2.0)
        d = jnp.pad(d, (0, npb - npad))
    k = npb // _S
    t2 = t.reshape(k, _S)
    d2 = d.reshape(k, _S)
    coarse = t2[:, -1].reshape(1, k)
    bsum = jnp.cumsum(jnp.sum(d2, axis=1))
    base = jnp.concatenate([jnp.zeros((1,), jnp.float32), bsum[:-1]])

    def trunc(x):
        u = jax.lax.bitcast_convert_type(x, jnp.int32)
        return jax.lax.bitcast_convert_type(u & jnp.int32(-65536), jnp.float32)

    def split3(x):
        h = trunc(x)
        r1 = x - h
        l1 = trunc(r1)
        l2 = r1 - l1
        return h, l1, l2

    def rnd(x):
        u = jax.lax.bitcast_convert_type(x, jnp.int32)
        return jax.lax.bitcast_convert_type(
            (u + jnp.int32(32768)) & jnp.int32(-65536), jnp.float32)

    th, tl1, tl2 = split3(t2)
    dh = rnd(d2)

    bsums = jnp.sum(d2, axis=1).at[k - 1].set(0.0).reshape(1, k)

    comb = jnp.concatenate([th, tl1, tl2, dh], axis=1).astype(jnp.bfloat16)
    comb_pos = dh.astype(jnp.bfloat16)
    return k, coarse, bsums, comb, comb_pos


def _tile_rows(flat, r):
    m = flat.shape[0]
    rows = max(1, _ceil_to(m, _LANES) // _LANES)
    rows_pad = _ceil_to(rows, r)
    total = rows_pad * _LANES
    if total != m:
        flat = jnp.pad(flat, (0, total - m))
    return flat.reshape(rows_pad, _LANES), rows_pad


def _device_mesh():
    devs = jax.devices()
    n = 2 if len(devs) >= 2 else 1
    return Mesh(np.array(devs[:n]), ("x",)), n


def kernel(table_pad, dlogp_pad, query, pos_items, seed_key):
    k, coarse, bsums, comb, comb_pos = _build_tables(table_pad, dlogp_pad)

    q_prefix = query.shape[:-1]
    num_queries = int(np.prod(q_prefix))
    num_neg = 32
    key = jax.random.wrap_key_data(seed_key)
    seeds = jax.random.uniform(key, (num_queries, num_neg), dtype=jnp.float32)

    mesh, ndev = _device_mesh()

    r = 64
    row_spec = pl.BlockSpec((r, _LANES), lambda i: (i, 0))
    m = num_queries * num_neg
    seeds2d, rows_pad = _tile_rows(seeds.reshape(-1), r * ndev)

    def sample_call(coarse_s, bs_s, comb_s, seeds_s):
        rows = seeds_s.shape[0]
        return pl.pallas_call(
            functools.partial(_sample_body, k=k, r=r, halves=8),
            out_shape=(jax.ShapeDtypeStruct((rows, _LANES), jnp.int32),
                       jax.ShapeDtypeStruct((rows, _LANES), jnp.float32)),
            grid=(rows // r,),
            in_specs=[pl.BlockSpec((1, k), lambda i: (0, 0)),
                      pl.BlockSpec((1, k), lambda i: (0, 0)),
                      pl.BlockSpec((k, 4 * _S), lambda i: (0, 0)),
                      row_spec],
            out_specs=[row_spec, row_spec],
            compiler_params=pltpu.CompilerParams(
                dimension_semantics=("parallel",),
                vmem_limit_bytes=64 * 1024 * 1024),
        )(coarse_s, bs_s, comb_s, seeds_s)

    if ndev > 1:
        sample_call = shard_map(
            sample_call, mesh=mesh,
            in_specs=(P(None, None), P(None, None), P(None, None),
                      P("x", None)),
            out_specs=(P("x", None), P("x", None)), check_rep=False)
    items2d, prob2d = sample_call(coarse, bsums, comb, seeds2d)
    neg_items = items2d.reshape(-1)[:m].reshape(*q_prefix, num_neg)
    neg_prob = prob2d.reshape(-1)[:m].reshape(*q_prefix, num_neg)

    rp = 16
    prow_spec = pl.BlockSpec((rp, _LANES), lambda i: (i, 0))
    mp = int(np.prod(pos_items.shape))
    pos2d, prows_pad = _tile_rows(pos_items.reshape(-1).astype(jnp.int32),
                                  rp * ndev)

    def pos_call(bs_s, comb_s, pos_s):
        rows = pos_s.shape[0]
        return pl.pallas_call(
            functools.partial(_pos_body, k=k, r=rp),
            out_shape=jax.ShapeDtypeStruct((rows, _LANES), jnp.float32),
            grid=(rows // rp,),
            in_specs=[pl.BlockSpec((1, k), lambda i: (0, 0)),
                      pl.BlockSpec((k, _S), lambda i: (0, 0)), prow_spec],
            out_specs=prow_spec,
            compiler_params=pltpu.CompilerParams(
                dimension_semantics=("parallel",),
                vmem_limit_bytes=64 * 1024 * 1024),
        )(bs_s, comb_s, pos_s)

    if ndev > 1:
        pos_call = shard_map(
            pos_call, mesh=mesh,
            in_specs=(P(None, None), P(None, None), P("x", None)),
            out_specs=P("x", None), check_rep=False)
    pprob2d = pos_call(bsums, comb_pos, pos2d)
    pos_prob = pprob2d.reshape(-1)[:mp].reshape(pos_items.shape)

    return pos_prob, neg_items, neg_prob
